# Initial kernel scaffold; baseline (speedup 1.0000x reference)
import jax
import jax.numpy as jnp
from jax.experimental import pallas as pl


def kernel(latents, embedding_weight):
    raise NotImplementedError("write your pallas kernel here")


# speedup vs baseline: 5.2080x; 5.2080x over previous
import functools

import jax
import jax.numpy as jnp
from jax import lax
from jax.experimental import pallas as pl
from jax.experimental.pallas import tpu as pltpu
from jax.experimental.pallas import tpu_sc as plsc

K = 8192
D = 32
N = 8192
BK = 256
NBK = K // BK
BETA = 0.25
LOSS_SCALE = (1.0 + BETA) / (N * D)


def _tc_argmin_body(z_ref, zsq_ref, e_ref, esq_ref, idx_ref, val_ref, loss_ref):
    j = pl.program_id(0)
    z = z_ref[...]
    e = e_ref[...]
    esq = esq_ref[...].reshape(1, BK)
    zsq = zsq_ref[...]
    mm = lax.dot_general(z, e, (((1,), (1,)), ((), ())),
                         preferred_element_type=jnp.float32)
    dist = (zsq + esq) - 2.0 * mm
    rowmin = jnp.min(dist, axis=1, keepdims=True)
    kiota = lax.broadcasted_iota(jnp.int32, (N, BK), 1) + j * BK
    cand = jnp.where(dist == rowmin, kiota, K)
    rowidx = jnp.min(cand, axis=1, keepdims=True)

    @pl.when(j == 0)
    def _():
        val_ref[...] = jnp.full((N, 128), jnp.inf, jnp.float32)
        idx_ref[...] = jnp.zeros((N, 128), jnp.int32)

    bv = val_ref[:, 0:1]
    bi = idx_ref[:, 0:1]
    upd = rowmin < bv
    nv = jnp.where(upd, rowmin, bv)
    ni = jnp.where(upd, rowidx, bi)
    val_ref[...] = jnp.broadcast_to(nv, (N, 128))
    idx_ref[...] = jnp.broadcast_to(ni, (N, 128))

    @pl.when(j == NBK - 1)
    def _():
        loss_ref[0, 0] = jnp.sum(nv) * jnp.float32(LOSS_SCALE)


def _tc_argmin(flat_z, zsq_col, e, esq3):
    return pl.pallas_call(
        _tc_argmin_body,
        grid=(NBK,),
        in_specs=[
            pl.BlockSpec((N, D), lambda j: (0, 0)),
            pl.BlockSpec((N, 1), lambda j: (0, 0)),
            pl.BlockSpec((BK, D), lambda j: (j, 0)),
            pl.BlockSpec((1, 1, BK), lambda j: (0, 0, j)),
        ],
        out_specs=[
            pl.BlockSpec((N, 128), lambda j: (0, 0)),
            pl.BlockSpec((N, 128), lambda j: (0, 0)),
            pl.BlockSpec(memory_space=pltpu.SMEM, block_shape=(1, 1),
                         index_map=lambda j: (0, 0)),
        ],
        out_shape=[
            jax.ShapeDtypeStruct((N, 128), jnp.int32),
            jax.ShapeDtypeStruct((N, 128), jnp.float32),
            jax.ShapeDtypeStruct((1, 1), jnp.float32),
        ],
    )(flat_z, zsq_col, e, esq3)


_SC_NC = 2
_SC_NS = 16
_NW = _SC_NC * _SC_NS
_BPW = N // _NW
_L = 16
_DP = 128


@functools.cache
def _sc_gather_fn():
    mesh = plsc.VectorSubcoreMesh(core_axis_name="c", subcore_axis_name="s",
                                  num_cores=_SC_NC, num_subcores=_SC_NS)

    @functools.partial(
        pl.kernel,
        out_type=jax.ShapeDtypeStruct((N, D), jnp.float32),
        mesh=mesh,
        scratch_types=[
            pltpu.VMEM((_BPW,), jnp.int32),
            pltpu.VMEM((_BPW, _DP), jnp.float32),
            pltpu.VMEM((_BPW, D), jnp.float32),
            pltpu.VMEM((_BPW, D), jnp.float32),
            pltpu.SemaphoreType.DMA,
        ],
    )
    def _sc_gather(table_hbm, idx_hbm, z_hbm, out_hbm,
                   idx_v, rows_v, z_v, out_v, sem):
        wid = lax.axis_index("s") * _SC_NC + lax.axis_index("c")
        base = wid * _BPW
        pltpu.sync_copy(idx_hbm.at[pl.ds(base, _BPW)], idx_v)
        gather = pltpu.async_copy(table_hbm.at[idx_v], rows_v, sem)
        pltpu.sync_copy(z_hbm.at[pl.ds(base, _BPW)], z_v)
        gather.wait()

        def body(r, carry):
            for h in range(D // _L):
                sl = pl.ds(h * _L, _L)
                q = rows_v[r, sl].astype(jnp.bfloat16).astype(jnp.float32)
                zz = z_v[r, sl]
                out_v[r, sl] = zz + (q - zz)
            return carry

        lax.fori_loop(0, _BPW, body, 0)
        pltpu.sync_copy(out_v, out_hbm.at[pl.ds(base, _BPW)])

    return _sc_gather


def kernel(latents, embedding_weight):
    flat_z = latents.reshape(N, D)
    zsq_col = jnp.sum(flat_z**2, axis=1, keepdims=True)
    esq3 = jnp.sum(embedding_weight**2, axis=1).reshape(1, 1, K)
    e_mm = embedding_weight.astype(jnp.bfloat16).astype(jnp.float32)
    idx_blocks, _, loss = _tc_argmin(flat_z, zsq_col, e_mm, esq3)
    idx = idx_blocks[:, 0]
    table_pad = jnp.pad(embedding_weight, ((0, 0), (0, _DP - D)))
    quantized = _sc_gather_fn()(table_pad, idx, flat_z)
    return quantized.reshape(latents.shape), loss.reshape(())
